# Initial kernel scaffold; baseline (speedup 1.0000x reference)
import jax
import jax.numpy as jnp
from jax.experimental import pallas as pl


def kernel(x, edge_index, edge_weight, W, b):
    raise NotImplementedError("write your pallas kernel here")


# speedup vs baseline: 4.4875x; 4.4875x over previous
import functools

import jax
import jax.numpy as jnp
from jax import lax
from jax.experimental import pallas as pl
from jax.experimental.pallas import tpu as pltpu
from jax.experimental.pallas import tpu_sc as plsc

NC = 2
NS = 16
LANES = 16
CHUNK = 128


def _mm_body(x_ref, w_ref, b_ref, o_ref):
    o_ref[...] = (
        lax.dot_general(
            x_ref[...], w_ref[...], (((1,), (1,)), ((), ())),
            preferred_element_type=jnp.float32,
        )
        + b_ref[...]
    )


def _linear(x, W, b):
    n, d_in = x.shape
    d_out = W.shape[0]
    blk = 1000
    grid = n // blk
    return pl.pallas_call(
        _mm_body,
        grid=(grid,),
        in_specs=[
            pl.BlockSpec((blk, d_in), lambda i: (i, 0)),
            pl.BlockSpec((d_out, d_in), lambda i: (0, 0)),
            pl.BlockSpec((1, d_out), lambda i: (0, 0)),
        ],
        out_specs=pl.BlockSpec((blk, d_out), lambda i: (i, 0)),
        out_shape=jax.ShapeDtypeStruct((n, d_out), jnp.float32),
    )(x, W, b.reshape(1, d_out))


def _add_body(a_ref, b_ref, o_ref):
    o_ref[...] = a_ref[...] + b_ref[...]


def _combine(p0, p1):
    n, d = p0.shape
    blk = 1000
    return pl.pallas_call(
        _add_body,
        grid=(n // blk,),
        in_specs=[
            pl.BlockSpec((blk, d), lambda i: (i, 0)),
            pl.BlockSpec((blk, d), lambda i: (i, 0)),
        ],
        out_specs=pl.BlockSpec((blk, d), lambda i: (i, 0)),
        out_shape=jax.ShapeDtypeStruct((n, d), jnp.float32),
    )(p0, p1)


def _make_sc_spmm(n, d, k):
    nw = NC * NS
    rpt = n // NS
    full = rpt // CHUNK
    rem = rpt % CHUNK
    nj = d // LANES

    mesh = plsc.VectorSubcoreMesh(
        core_axis_name="c", subcore_axis_name="s",
        num_cores=NC, num_subcores=NS,
    )

    @functools.partial(
        pl.kernel,
        out_type=jax.ShapeDtypeStruct((NC * n, d), jnp.float32),
        mesh=mesh,
        scratch_types=[
            pltpu.VMEM((k, CHUNK), jnp.int32),
            pltpu.VMEM((k, CHUNK), jnp.int32),
            pltpu.VMEM((k, CHUNK), jnp.float32),
            pltpu.VMEM((CHUNK, d), jnp.float32),
            pltpu.VMEM_SHARED((n, d), jnp.float32),
            pltpu.SemaphoreType.DMA,
        ],
    )
    def sc_kernel(h_hbm, src_hbm, dst_hbm, w_hbm, out_hbm,
                  src_v, dst_v, w_v, rows_v, acc_sh, sem):
        cid = lax.axis_index("c")
        sid = lax.axis_index("s")
        wid = sid * NC + cid

        pltpu.sync_copy(src_hbm.at[wid], src_v)
        pltpu.sync_copy(dst_hbm.at[wid], dst_v)
        pltpu.sync_copy(w_hbm.at[wid], w_v)

        zero = jnp.zeros((LANES,), jnp.float32)

        def _zrow(i, carry):
            for j in range(nj):
                rows_v[i, pl.ds(j * LANES, LANES)] = zero
            return carry

        lax.fori_loop(0, CHUNK, _zrow, 0)

        zbase = sid * rpt
        for c0 in range(full):
            pltpu.sync_copy(rows_v, acc_sh.at[pl.ds(zbase + c0 * CHUNK, CHUNK)])
        if rem:
            pltpu.sync_copy(rows_v.at[pl.ds(0, rem)],
                            acc_sh.at[pl.ds(zbase + full * CHUNK, rem)])
        plsc.subcore_barrier()

        def _chunk(g, carry):
            pltpu.async_copy(h_hbm.at[src_v.at[g]], rows_v, sem).wait()

            def _scale(t, c2):
                wvec = w_v[g, pl.ds(t * LANES, LANES)]
                for l in range(LANES):
                    w = wvec[l]
                    ei = t * LANES + l
                    for j in range(nj):
                        sl = pl.ds(j * LANES, LANES)
                        rows_v[ei, sl] = rows_v[ei, sl] * w
                return c2

            lax.fori_loop(0, CHUNK // LANES, _scale, 0)
            pltpu.sync_copy(rows_v, acc_sh.at[dst_v.at[g]], add=True)
            return carry

        lax.fori_loop(0, k, _chunk, 0)
        plsc.subcore_barrier()

        obase = cid * n + sid * rpt
        for c0 in range(full):
            pltpu.sync_copy(acc_sh.at[pl.ds(zbase + c0 * CHUNK, CHUNK)],
                            out_hbm.at[pl.ds(obase + c0 * CHUNK, CHUNK)])
        if rem:
            pltpu.sync_copy(acc_sh.at[pl.ds(zbase + full * CHUNK, rem)],
                            out_hbm.at[pl.ds(obase + full * CHUNK, rem)])

    return sc_kernel


def kernel(x, edge_index, edge_weight, W, b):
    n, _ = x.shape
    d = W.shape[0]
    e = edge_weight.shape[0]

    h = _linear(x, W, b)

    nw = NC * NS
    per_w = -(-e // (nw * CHUNK)) * CHUNK
    e_pad = per_w * nw
    k = per_w // CHUNK

    dst = edge_index[0].astype(jnp.int32)
    src = edge_index[1].astype(jnp.int32)
    w = edge_weight.astype(jnp.float32)
    pad = e_pad - e
    if pad:
        dst = jnp.pad(dst, (0, pad))
        src = jnp.pad(src, (0, pad))
        w = jnp.pad(w, (0, pad))
    src3 = src.reshape(nw, k, CHUNK)
    dst3 = dst.reshape(nw, k, CHUNK)
    w3 = w.reshape(nw, k, CHUNK)

    n_pad = -(-n // (NS * 8)) * (NS * 8)
    partials = _make_sc_spmm(n_pad, d, k)(h, src3, dst3, w3)
    return _combine(partials[:n], partials[n_pad:n_pad + n])
